# speedup vs baseline: 40.2726x; 40.2726x over previous
import functools

import jax
import jax.numpy as jnp
from jax.experimental import pallas as pl

_LANES = 128
_BQ = 4096


def _body(vecs_ref, queue_ref, ids_ref, qids_ref, valid_ref,
          oq_ref, oids_ref, oval_ref, *, nvb):
    i = pl.program_id(0)

    @pl.when(i < nvb)
    def _():
        oq_ref[...] = vecs_ref[...]
        oids_ref[...] = ids_ref[...]
        oval_ref[...] = jnp.ones_like(oval_ref)

    @pl.when(i >= nvb)
    def _():
        oq_ref[...] = queue_ref[...]
        oids_ref[...] = qids_ref[...]
        oval_ref[...] = valid_ref[...]


def kernel(vecs, ids, queue, queue_ids, valid):
    batch, dim = vecs.shape
    k = queue.shape[0]
    bq = _BQ
    nvb = batch // bq
    grid = k // bq
    rows = bq // _LANES

    ids2d = ids.reshape(batch // _LANES, _LANES)
    qids2d = queue_ids.reshape(k // _LANES, _LANES)
    valid2d = valid.astype(jnp.int8).reshape(k // _LANES, _LANES)

    body = functools.partial(_body, nvb=nvb)

    oq, oids2d, oval2d = pl.pallas_call(
        body,
        grid=(grid,),
        in_specs=[
            pl.BlockSpec((bq, dim), lambda i: (jnp.minimum(i, nvb - 1), 0)),
            pl.BlockSpec((bq, dim), lambda i: (jnp.maximum(i, nvb), 0)),
            pl.BlockSpec((rows, _LANES), lambda i: (jnp.minimum(i, nvb - 1), 0)),
            pl.BlockSpec((rows, _LANES), lambda i: (jnp.maximum(i, nvb), 0)),
            pl.BlockSpec((rows, _LANES), lambda i: (jnp.maximum(i, nvb), 0)),
        ],
        out_specs=[
            pl.BlockSpec((bq, dim), lambda i: (i, 0)),
            pl.BlockSpec((rows, _LANES), lambda i: (i, 0)),
            pl.BlockSpec((rows, _LANES), lambda i: (i, 0)),
        ],
        out_shape=[
            jax.ShapeDtypeStruct((k, dim), queue.dtype),
            jax.ShapeDtypeStruct((k // _LANES, _LANES), queue_ids.dtype),
            jax.ShapeDtypeStruct((k // _LANES, _LANES), jnp.int8),
        ],
    )(vecs, queue, ids2d, qids2d, valid2d)

    return (oq, oids2d.reshape(k), oval2d.reshape(k).astype(jnp.bool_))


# speedup vs baseline: 43.9438x; 1.0912x over previous
import functools

import jax
import jax.numpy as jnp
from jax.experimental import pallas as pl

_LANES = 128
_BQ = 8192


def _body(vecs_ref, queue_ref, ids_ref, qids_ref, valid_ref,
          oq_ref, oids_ref, oval_ref, *, nvb):
    i = pl.program_id(0)

    @pl.when(i < nvb)
    def _():
        oq_ref[...] = vecs_ref[...]
        oids_ref[...] = ids_ref[...]
        oval_ref[...] = jnp.ones_like(oval_ref)

    @pl.when(i >= nvb)
    def _():
        oq_ref[...] = queue_ref[...]
        oids_ref[...] = qids_ref[...]
        oval_ref[...] = valid_ref[...]


def kernel(vecs, ids, queue, queue_ids, valid):
    batch, dim = vecs.shape
    k = queue.shape[0]
    bq = _BQ
    nvb = batch // bq
    grid = k // bq
    rows = bq // _LANES

    ids2d = ids.reshape(batch // _LANES, _LANES)
    qids2d = queue_ids.reshape(k // _LANES, _LANES)
    valid2d = valid.astype(jnp.int8).reshape(k // _LANES, _LANES)

    body = functools.partial(_body, nvb=nvb)

    oq, oids2d, oval2d = pl.pallas_call(
        body,
        grid=(grid,),
        in_specs=[
            pl.BlockSpec((bq, dim), lambda i: (jnp.minimum(i, nvb - 1), 0)),
            pl.BlockSpec((bq, dim), lambda i: (jnp.maximum(i, nvb), 0)),
            pl.BlockSpec((rows, _LANES), lambda i: (jnp.minimum(i, nvb - 1), 0)),
            pl.BlockSpec((rows, _LANES), lambda i: (jnp.maximum(i, nvb), 0)),
            pl.BlockSpec((rows, _LANES), lambda i: (jnp.maximum(i, nvb), 0)),
        ],
        out_specs=[
            pl.BlockSpec((bq, dim), lambda i: (i, 0)),
            pl.BlockSpec((rows, _LANES), lambda i: (i, 0)),
            pl.BlockSpec((rows, _LANES), lambda i: (i, 0)),
        ],
        out_shape=[
            jax.ShapeDtypeStruct((k, dim), queue.dtype),
            jax.ShapeDtypeStruct((k // _LANES, _LANES), queue_ids.dtype),
            jax.ShapeDtypeStruct((k // _LANES, _LANES), jnp.int8),
        ],
    )(vecs, queue, ids2d, qids2d, valid2d)

    return (oq, oids2d.reshape(k), oval2d.reshape(k).astype(jnp.bool_))
